# Initial kernel scaffold; baseline (speedup 1.0000x reference)
import functools

import jax
import jax.numpy as jnp
from jax import lax
from jax.experimental import pallas as pl
from jax.experimental.pallas import tpu as pltpu
from jax.experimental.pallas import tpu_sc as plsc

_NC = 2
_NS = 16
_NW = _NC * _NS


def _mm_block(x_ref, w_ref, o_ref):
    o_ref[...] = jnp.dot(x_ref[...], w_ref[...], preferred_element_type=jnp.float32)


def _matmul(x, w, block_m):
    m, k = x.shape
    _, n = w.shape
    return pl.pallas_call(
        _mm_block,
        grid=(m // block_m,),
        in_specs=[
            pl.BlockSpec((block_m, k), lambda i: (i, 0)),
            pl.BlockSpec((k, n), lambda i: (0, 0)),
        ],
        out_specs=pl.BlockSpec((block_m, n), lambda i: (i, 0)),
        out_shape=jax.ShapeDtypeStruct((m, n), jnp.float32),
    )(x, w)


def _addmm_block(p_ref, w_ref, o_ref):
    h = p_ref[0] + p_ref[1]
    o_ref[...] = jnp.dot(h, w_ref[...], preferred_element_type=jnp.float32)


def _add_matmul(p, w, block_m):
    _, m, k = p.shape
    _, n = w.shape
    return pl.pallas_call(
        _addmm_block,
        grid=(m // block_m,),
        in_specs=[
            pl.BlockSpec((2, block_m, k), lambda i: (0, i, 0)),
            pl.BlockSpec((k, n), lambda i: (0, 0)),
        ],
        out_specs=pl.BlockSpec((block_m, n), lambda i: (i, 0)),
        out_shape=jax.ShapeDtypeStruct((m, n), jnp.float32),
    )(p, w)


def _add2_block(p_ref, o_ref):
    o_ref[...] = p_ref[0] + p_ref[1]


def _add2(p, block_m):
    _, m, n = p.shape
    return pl.pallas_call(
        _add2_block,
        grid=(m // block_m,),
        in_specs=[pl.BlockSpec((2, block_m, n), lambda i: (0, i, 0))],
        out_specs=pl.BlockSpec((block_m, n), lambda i: (i, 0)),
        out_shape=jax.ShapeDtypeStruct((m, n), jnp.float32),
    )(p)


def _sc_gather_scatter(h, src, dst, zeros, n_nodes, chunk):
    e = src.shape[0]
    d = h.shape[1]
    ew = e // _NW
    nch = ew // chunk
    rpt = n_nodes // _NS

    mesh = plsc.VectorSubcoreMesh(
        core_axis_name="c", subcore_axis_name="s", num_cores=_NC, num_subcores=_NS
    )

    @functools.partial(
        pl.kernel,
        out_type=jax.ShapeDtypeStruct((_NC, n_nodes, d), jnp.float32),
        mesh=mesh,
        scratch_types=[
            pltpu.VMEM((chunk,), jnp.int32),
            pltpu.VMEM((chunk,), jnp.int32),
            pltpu.VMEM((chunk, d), jnp.float32),
            pltpu.VMEM_SHARED((n_nodes, d), jnp.float32),
            pltpu.SemaphoreType.DMA,
        ],
    )
    def edge_pass(h_hbm, src_hbm, dst_hbm, z_hbm, out_hbm, sidx, didx, rows, acc, sem):
        cid = lax.axis_index("c")
        sid = lax.axis_index("s")
        wid = sid * _NC + cid
        stripe = pl.ds(sid * rpt, rpt)
        pltpu.sync_copy(z_hbm.at[stripe], acc.at[stripe])
        plsc.subcore_barrier()

        def step(j, carry):
            base = wid * ew + j * chunk
            pltpu.sync_copy(src_hbm.at[pl.ds(base, chunk)], sidx)
            pltpu.sync_copy(dst_hbm.at[pl.ds(base, chunk)], didx)
            pltpu.async_copy(h_hbm.at[sidx], rows, sem).wait()
            pltpu.sync_copy(rows, acc.at[didx], add=True)
            return carry

        lax.fori_loop(0, nch, step, 0)
        plsc.subcore_barrier()
        pltpu.sync_copy(acc.at[stripe], out_hbm.at[cid, stripe])

    return edge_pass(h, src, dst, zeros)


def kernel(x, edge_index, W1, W2):
    n = x.shape[0]
    src = edge_index[0].astype(jnp.int32)
    dst = edge_index[1].astype(jnp.int32)
    z1 = jnp.zeros((n, W1.shape[1]), jnp.float32)
    z2 = jnp.zeros((n, W2.shape[1]), jnp.float32)

    h1p = _matmul(x, W1, block_m=1000)
    p1 = _sc_gather_scatter(h1p, src, dst, z1, n, chunk=80)
    h2p = _add_matmul(p1, W2, block_m=1000)
    p2 = _sc_gather_scatter(h2p, src, dst, z2, n, chunk=80)
    return _add2(p2, block_m=1000)


# speedup vs baseline: 5.4591x; 5.4591x over previous
import functools

import jax
import jax.numpy as jnp
from jax import lax
from jax.experimental import pallas as pl
from jax.experimental.pallas import tpu as pltpu
from jax.experimental.pallas import tpu_sc as plsc

_NC = 2
_NS = 16
_NW = _NC * _NS


def _mm_block(x_ref, w_ref, o_ref):
    o_ref[...] = jnp.dot(x_ref[...], w_ref[...], preferred_element_type=jnp.float32)


def _matmul(x, w, block_m):
    m, k = x.shape
    _, n = w.shape
    return pl.pallas_call(
        _mm_block,
        grid=(m // block_m,),
        in_specs=[
            pl.BlockSpec((block_m, k), lambda i: (i, 0)),
            pl.BlockSpec((k, n), lambda i: (0, 0)),
        ],
        out_specs=pl.BlockSpec((block_m, n), lambda i: (i, 0)),
        out_shape=jax.ShapeDtypeStruct((m, n), jnp.float32),
    )(x, w)


def _addmm_block(p_ref, w_ref, o_ref):
    h = p_ref[0] + p_ref[1]
    o_ref[...] = jnp.dot(h, w_ref[...], preferred_element_type=jnp.float32)


def _add_matmul(p, w, block_m, m):
    _, _, k = p.shape
    _, n = w.shape
    return pl.pallas_call(
        _addmm_block,
        grid=(m // block_m,),
        in_specs=[
            pl.BlockSpec((2, block_m, k), lambda i: (0, i, 0)),
            pl.BlockSpec((k, n), lambda i: (0, 0)),
        ],
        out_specs=pl.BlockSpec((block_m, n), lambda i: (i, 0)),
        out_shape=jax.ShapeDtypeStruct((m, n), jnp.float32),
    )(p, w)


def _add2_block(p_ref, o_ref):
    o_ref[...] = p_ref[0] + p_ref[1]


def _add2(p, block_m, m):
    _, _, n = p.shape
    return pl.pallas_call(
        _add2_block,
        grid=(m // block_m,),
        in_specs=[pl.BlockSpec((2, block_m, n), lambda i: (0, i, 0))],
        out_specs=pl.BlockSpec((block_m, n), lambda i: (i, 0)),
        out_shape=jax.ShapeDtypeStruct((m, n), jnp.float32),
    )(p)


def _sc_gather_scatter(h, src, dst, n_nodes, chunk):
    e = src.shape[0]
    d = h.shape[1]
    ew = e // _NW
    nch = ew // chunk
    rpt = -(-n_nodes // (_NS * 8)) * 8
    npad = rpt * _NS

    mesh = plsc.VectorSubcoreMesh(
        core_axis_name="c", subcore_axis_name="s", num_cores=_NC, num_subcores=_NS
    )

    @functools.partial(
        pl.kernel,
        out_type=jax.ShapeDtypeStruct((_NC, npad, d), jnp.float32),
        mesh=mesh,
        scratch_types=[
            pltpu.VMEM((chunk,), jnp.int32),
            pltpu.VMEM((chunk,), jnp.int32),
            pltpu.VMEM((chunk, d), jnp.float32),
            pltpu.VMEM_SHARED((npad, d), jnp.float32),
            pltpu.SemaphoreType.DMA,
        ],
        compiler_params=pltpu.CompilerParams(use_tc_tiling_on_sc=False),
    )
    def edge_pass(h_hbm, src_hbm, dst_hbm, z_hbm, out_hbm, sidx, didx, rows, acc, sem):
        cid = lax.axis_index("c")
        sid = lax.axis_index("s")
        wid = sid * _NC + cid
        stripe = pl.ds(sid * rpt, rpt)
        pltpu.sync_copy(z_hbm.at[stripe], acc.at[stripe])
        plsc.subcore_barrier()

        def step(j, carry):
            base = wid * ew + j * chunk
            pltpu.sync_copy(src_hbm.at[pl.ds(base, chunk)], sidx)
            pltpu.sync_copy(dst_hbm.at[pl.ds(base, chunk)], didx)
            pltpu.async_copy(h_hbm.at[sidx], rows, sem).wait()
            pltpu.sync_copy(rows, acc.at[didx], add=True)
            return carry

        lax.fori_loop(0, nch, step, 0)
        plsc.subcore_barrier()
        pltpu.sync_copy(acc.at[stripe], out_hbm.at[cid, stripe])

    zeros = jnp.zeros((npad, d), jnp.float32)
    return edge_pass(h, src, dst, zeros)


def kernel(x, edge_index, W1, W2):
    n = x.shape[0]
    src = edge_index[0].astype(jnp.int32)
    dst = edge_index[1].astype(jnp.int32)

    h1p = _matmul(x, W1, block_m=1000)
    p1 = _sc_gather_scatter(h1p, src, dst, n, chunk=80)
    h2p = _add_matmul(p1, W2, block_m=1000, m=n)
    p2 = _sc_gather_scatter(h2p, src, dst, n, chunk=80)
    return _add2(p2, block_m=1000, m=n)


# speedup vs baseline: 16.3212x; 2.9897x over previous
import functools
import math

import jax
import jax.numpy as jnp
from jax import lax
from jax.experimental import pallas as pl
from jax.experimental.pallas import tpu as pltpu
from jax.experimental.pallas import tpu_sc as plsc

_NC = 2
_NS = 16
_NW = _NC * _NS


def _mm_block(x_ref, w_ref, o_ref):
    o_ref[...] = jnp.dot(x_ref[...], w_ref[...], preferred_element_type=jnp.float32)


def _matmul(x, w, block_m):
    m, k = x.shape
    _, n = w.shape
    return pl.pallas_call(
        _mm_block,
        grid=(m // block_m,),
        in_specs=[
            pl.BlockSpec((block_m, k), lambda i: (i, 0)),
            pl.BlockSpec((k, n), lambda i: (0, 0)),
        ],
        out_specs=pl.BlockSpec((block_m, n), lambda i: (i, 0)),
        out_shape=jax.ShapeDtypeStruct((m, n), jnp.float32),
    )(x, w)


def _addmm_block(p_ref, w_ref, o_ref):
    h = p_ref[0] + p_ref[1]
    o_ref[...] = jnp.dot(h, w_ref[...], preferred_element_type=jnp.float32)


def _add_matmul(p, w, block_m, m):
    _, _, k = p.shape
    _, n = w.shape
    return pl.pallas_call(
        _addmm_block,
        grid=(m // block_m,),
        in_specs=[
            pl.BlockSpec((2, block_m, k), lambda i: (0, i, 0)),
            pl.BlockSpec((k, n), lambda i: (0, 0)),
        ],
        out_specs=pl.BlockSpec((block_m, n), lambda i: (i, 0)),
        out_shape=jax.ShapeDtypeStruct((m, n), jnp.float32),
    )(p, w)


def _add2_block(p_ref, o_ref):
    o_ref[...] = p_ref[0] + p_ref[1]


def _add2(p):
    _, m, n = p.shape
    return pl.pallas_call(
        _add2_block,
        in_specs=[pl.BlockSpec((2, m, n), lambda: (0, 0, 0))],
        out_specs=pl.BlockSpec((m, n), lambda: (0, 0)),
        out_shape=jax.ShapeDtypeStruct((m, n), jnp.float32),
    )(p)


def _sc_gather_scatter(h, src, dst, chunk, nbuf, rpt):
    e = src.shape[0]
    d = h.shape[1]
    ew = e // _NW
    nch = ew // chunk
    npad = rpt * _NS

    mesh = plsc.VectorSubcoreMesh(
        core_axis_name="c", subcore_axis_name="s", num_cores=_NC, num_subcores=_NS
    )

    @functools.partial(
        pl.kernel,
        out_type=jax.ShapeDtypeStruct((_NC, npad, d), jnp.float32),
        mesh=mesh,
        scratch_types=[
            pltpu.VMEM((nch, chunk), jnp.int32),
            pltpu.VMEM((nch, chunk), jnp.int32),
            [pltpu.VMEM((chunk, d), jnp.float32) for _ in range(nbuf)],
            pltpu.VMEM_SHARED((npad, d), jnp.float32),
            [pltpu.SemaphoreType.DMA for _ in range(nbuf)],
        ],
        compiler_params=pltpu.CompilerParams(use_tc_tiling_on_sc=False),
    )
    def edge_pass(h_hbm, src_hbm, dst_hbm, z_hbm, out_hbm,
                  sidx, didx, rows, acc, sems):
        cid = lax.axis_index("c")
        sid = lax.axis_index("s")
        wid = sid * _NC + cid
        stripe = pl.ds(sid * rpt, rpt)
        pltpu.sync_copy(src_hbm.at[wid], sidx)
        pltpu.sync_copy(dst_hbm.at[wid], didx)
        pltpu.sync_copy(z_hbm.at[stripe], acc.at[stripe])
        plsc.subcore_barrier()

        for b in range(nbuf - 1):
            pltpu.async_copy(h_hbm.at[sidx.at[b]], rows[b], sems[b])

        def step(j, carry):
            nxt = j + (nbuf - 1)
            for b in range(nbuf):
                @pl.when(jnp.logical_and(nxt < nch, nxt % nbuf == b))
                def _(b=b):
                    pltpu.async_copy(h_hbm.at[sidx.at[nxt]], rows[b], sems[b])

            for b in range(nbuf):
                @pl.when(j % nbuf == b)
                def _(b=b):
                    pltpu.make_async_copy(h_hbm.at[sidx.at[j]], rows[b], sems[b]).wait()
                    pltpu.sync_copy(rows[b], acc.at[didx.at[j]], add=True)

            return carry

        lax.fori_loop(0, nch, step, 0)
        plsc.subcore_barrier()
        pltpu.sync_copy(acc.at[stripe], out_hbm.at[cid, stripe])

    zeros = jnp.zeros((npad, d), jnp.float32)
    src_r = src.reshape(_NW, nch, chunk)
    dst_r = dst.reshape(_NW, nch, chunk)
    return edge_pass(h, src_r, dst_r, zeros)


def kernel(x, edge_index, W1, W2):
    n = x.shape[0]
    d2 = W2.shape[1]
    src = edge_index[0].astype(jnp.int32)
    dst = edge_index[1].astype(jnp.int32)

    e = src.shape[0]
    epad = -(-e // (_NW * 128 * 8)) * (_NW * 128 * 8)
    npad_e = epad - e
    rpt1 = -(-n // (_NS * 8)) * 8
    spare = rpt1 * _NS - n
    if npad_e:
        pad_ids = jnp.arange(npad_e, dtype=jnp.int32)
        src = jnp.concatenate([src, pad_ids % n])
        dst = jnp.concatenate([dst, n + pad_ids % spare])

    g = 128 // math.gcd(d2, 128)
    rpt2 = -(-n // (_NS * 8 * g)) * 8 * g

    h1p = _matmul(x, W1, block_m=2000)
    p1 = _sc_gather_scatter(h1p, src, dst, 64, 3, rpt1)
    h2p = _add_matmul(p1, W2, block_m=2000, m=n)
    p2 = _sc_gather_scatter(h2p, src, dst, 128, 4, rpt2)
    npad2 = rpt2 * _NS
    p2r = p2.reshape(2, npad2 * d2 // 128, 128)
    s = _add2(p2r)
    return s.reshape(npad2, d2)[:n]
